# Initial kernel scaffold; baseline (speedup 1.0000x reference)
import jax
import jax.numpy as jnp
from jax.experimental import pallas as pl


def kernel(i, j, row_embed, col_embed):
    raise NotImplementedError("write your pallas kernel here")


# speedup vs baseline: 1.3303x; 1.3303x over previous
import functools

import jax
import jax.numpy as jnp
from jax import lax
from jax.experimental import pallas as pl
from jax.experimental.pallas import tpu as pltpu
from jax.experimental.pallas import tpu_sc as plsc

B_TOT = 64 * 32 * 32
D = 256
NC, NS = 2, 16
NW = NC * NS
BPW = B_TOT // NW
CH = 128
NCHUNK = BPW // CH

_mesh = plsc.VectorSubcoreMesh(core_axis_name="c", subcore_axis_name="s")


@functools.partial(
    pl.kernel,
    mesh=_mesh,
    out_type=jax.ShapeDtypeStruct((B_TOT, 2, D), jnp.float32),
    scratch_types=[
        pltpu.VMEM((BPW,), jnp.int32),
        pltpu.VMEM((BPW,), jnp.int32),
        pltpu.VMEM((CH, D), jnp.float32),
        pltpu.VMEM((CH, D), jnp.float32),
        pltpu.SemaphoreType.DMA,
        pltpu.SemaphoreType.DMA,
    ],
)
def _emb_lookup(i_hbm, j_hbm, col_hbm, row_hbm, out_hbm, i_v, j_v, bi, bj, s0, s1):
    wid = lax.axis_index("s") * NC + lax.axis_index("c")
    base = wid * BPW
    pltpu.sync_copy(i_hbm.at[pl.ds(base, BPW)], i_v)
    pltpu.sync_copy(j_hbm.at[pl.ds(base, BPW)], j_v)
    for c in range(NCHUNK):
        off = c * CH
        gi = pltpu.async_copy(col_hbm.at[i_v.at[pl.ds(off, CH)]], bi, s0)
        gj = pltpu.async_copy(row_hbm.at[j_v.at[pl.ds(off, CH)]], bj, s1)
        gi.wait()
        pltpu.sync_copy(bi, out_hbm.at[pl.ds(base + off, CH), 0])
        gj.wait()
        pltpu.sync_copy(bj, out_hbm.at[pl.ds(base + off, CH), 1])


def kernel(i, j, row_embed, col_embed):
    out = _emb_lookup(i.reshape(-1), j.reshape(-1), col_embed, row_embed)
    return out.reshape(64, 32, 32, 2 * D)


# speedup vs baseline: 1.3445x; 1.0107x over previous
import functools

import jax
import jax.numpy as jnp
from jax import lax
from jax.experimental import pallas as pl
from jax.experimental.pallas import tpu as pltpu
from jax.experimental.pallas import tpu_sc as plsc

B_TOT = 64 * 32 * 32
D = 256
NC, NS = 2, 16
NW = NC * NS
BPW = B_TOT // NW
CH = 64
NCHUNK = BPW // CH

_mesh = plsc.VectorSubcoreMesh(core_axis_name="c", subcore_axis_name="s")


@functools.partial(
    pl.kernel,
    mesh=_mesh,
    out_type=jax.ShapeDtypeStruct((B_TOT, 2, D), jnp.float32),
    scratch_types=[
        pltpu.VMEM((BPW,), jnp.int32),
        pltpu.VMEM((BPW,), jnp.int32),
        pltpu.VMEM((CH, D), jnp.float32),
        pltpu.VMEM((CH, D), jnp.float32),
        pltpu.VMEM((CH, D), jnp.float32),
        pltpu.VMEM((CH, D), jnp.float32),
        pltpu.SemaphoreType.DMA,
        pltpu.SemaphoreType.DMA,
        pltpu.SemaphoreType.DMA,
        pltpu.SemaphoreType.DMA,
    ],
)
def _emb_lookup(i_hbm, j_hbm, col_hbm, row_hbm, out_hbm,
                i_v, j_v, bi0, bi1, bj0, bj1, g0, g1, w0, w1):
    wid = lax.axis_index("s") * NC + lax.axis_index("c")
    base = wid * BPW
    pltpu.sync_copy(i_hbm.at[pl.ds(base, BPW)], i_v)
    pltpu.sync_copy(j_hbm.at[pl.ds(base, BPW)], j_v)

    ibufs, jbufs, gsems, wsems = (bi0, bi1), (bj0, bj1), (g0, g1), (w0, w1)

    def fire(c):
        nb = c % 2
        off = c * CH
        di = pltpu.async_copy(col_hbm.at[i_v.at[pl.ds(off, CH)]],
                              ibufs[nb], gsems[nb])
        dj = pltpu.async_copy(row_hbm.at[j_v.at[pl.ds(off, CH)]],
                              jbufs[nb], gsems[nb])
        return di, dj

    gathers = fire(0)
    writes = [None, None]
    for c in range(NCHUNK):
        nb = c % 2
        di, dj = gathers
        di.wait()
        dj.wait()
        wi = pltpu.async_copy(
            ibufs[nb], out_hbm.at[pl.ds(base + c * CH, CH), 0], wsems[nb])
        wj = pltpu.async_copy(
            jbufs[nb], out_hbm.at[pl.ds(base + c * CH, CH), 1], wsems[nb])
        writes[nb] = (wi, wj)
        if c + 1 < NCHUNK:
            prev = writes[(c + 1) % 2]
            if prev is not None:
                prev[0].wait()
                prev[1].wait()
            gathers = fire(c + 1)
    writes[(NCHUNK - 1) % 2][0].wait()
    writes[(NCHUNK - 1) % 2][1].wait()


def kernel(i, j, row_embed, col_embed):
    out = _emb_lookup(i.reshape(-1), j.reshape(-1), col_embed, row_embed)
    return out.reshape(64, 32, 32, 2 * D)
